# baseline (speedup 1.0000x reference)
import functools

import jax
import jax.numpy as jnp
from jax.experimental import pallas as pl

D_MODEL = 4096
NUM_EXPERTS = 64
TOP_K = 8
LB_COEF = 0.01

_NEG = -1e30


def _router_kernel(x_ref, w_ref, probs_ref, idx_ref, acc_ref, loss_ref,
                   *, num_blocks, total_tokens):
    i = pl.program_id(0)

    logits = jax.lax.dot_general(
        x_ref[...], w_ref[...],
        dimension_numbers=(((1,), (0,)), ((), ())),
        preferred_element_type=jnp.float32)

    t = logits.shape[0]
    lt = logits.T
    iota = jax.lax.broadcasted_iota(jnp.int32, (NUM_EXPERTS, t), 0)

    work = lt
    vals = []
    idxs = []
    for _ in range(TOP_K):
        m = jnp.max(work, axis=0, keepdims=True)
        hit = work == m
        ix = jnp.min(jnp.where(hit, iota, NUM_EXPERTS), axis=0,
                     keepdims=True)
        vals.append(m)
        idxs.append(ix)
        work = jnp.where(iota == ix, _NEG, work)

    top_vals = jnp.concatenate(vals, axis=0)
    top_idx = jnp.concatenate(idxs, axis=0)

    e = jnp.exp(top_vals - top_vals[0:1])
    probs_ref[...] = (e / jnp.sum(e, axis=0, keepdims=True)).T
    idx_ref[...] = top_idx.T

    fe = jnp.exp(lt - top_vals[0:1])
    rp = fe / jnp.sum(fe, axis=0, keepdims=True)
    rowsum = jnp.sum(rp, axis=1, keepdims=True)

    @pl.when(i == 0)
    def _init():
        acc_ref[...] = rowsum

    @pl.when(i > 0)
    def _accum():
        acc_ref[...] += rowsum

    @pl.when(i == num_blocks - 1)
    def _finalize():
        ep = acc_ref[...] * (1.0 / total_tokens)
        loss_ref[...] = LB_COEF * jnp.sum(
            ep * jnp.log(ep + 1e-8), keepdims=True)[:, :1]


@functools.partial(jax.jit, static_argnames=())
def kernel(x, W):
    b, s, d = x.shape
    n_tok = b * s
    block_t = 512
    num_blocks = n_tok // block_t
    x2 = x.reshape(n_tok, d)
    w_bf = W.astype(jnp.bfloat16)

    grid_spec = pl.GridSpec(
        grid=(num_blocks,),
        in_specs=[
            pl.BlockSpec((block_t, d), lambda i: (i, 0)),
            pl.BlockSpec((d, NUM_EXPERTS), lambda i: (0, 0)),
        ],
        out_specs=[
            pl.BlockSpec((block_t, TOP_K), lambda i: (i, 0)),
            pl.BlockSpec((block_t, TOP_K), lambda i: (i, 0)),
            pl.BlockSpec((NUM_EXPERTS, 1), lambda i: (0, 0)),
            pl.BlockSpec((1, 1), lambda i: (0, 0)),
        ],
    )

    probs, idx, _, loss = pl.pallas_call(
        functools.partial(_router_kernel, num_blocks=num_blocks,
                          total_tokens=n_tok),
        grid_spec=grid_spec,
        out_shape=[
            jax.ShapeDtypeStruct((n_tok, TOP_K), jnp.float32),
            jax.ShapeDtypeStruct((n_tok, TOP_K), jnp.int32),
            jax.ShapeDtypeStruct((NUM_EXPERTS, 1), jnp.float32),
            jax.ShapeDtypeStruct((1, 1), jnp.float32),
        ],
    )(x2, w_bf)

    return (probs.reshape(b, s, TOP_K), idx.reshape(b, s, TOP_K),
            loss.reshape(()))


# speedup vs baseline: 1.0568x; 1.0568x over previous
import functools

import jax
import jax.numpy as jnp
from jax.experimental import pallas as pl

D_MODEL = 4096
NUM_EXPERTS = 64
TOP_K = 8
LB_COEF = 0.01

_NEG = -1e30


def _router_kernel(x_ref, w_ref, probs_ref, idx_ref, acc_ref, loss_ref,
                   *, num_blocks, total_tokens):
    i = pl.program_id(0)

    logits = jax.lax.dot_general(
        x_ref[...], w_ref[...],
        dimension_numbers=(((1,), (0,)), ((), ())),
        preferred_element_type=jnp.float32)

    t = logits.shape[0]
    lt = logits.T
    iota = jax.lax.broadcasted_iota(jnp.int32, (NUM_EXPERTS, t), 0)

    work = lt
    vals = []
    idxs = []
    for _ in range(TOP_K):
        m = jnp.max(work, axis=0, keepdims=True)
        hit = work == m
        ix = jnp.min(jnp.where(hit, iota, NUM_EXPERTS), axis=0,
                     keepdims=True)
        vals.append(m)
        idxs.append(ix)
        work = jnp.where(iota == ix, _NEG, work)

    top_vals = jnp.concatenate(vals, axis=0)
    top_idx = jnp.concatenate(idxs, axis=0)

    e = jnp.exp(top_vals - top_vals[0:1])
    probs_ref[...] = (e / jnp.sum(e, axis=0, keepdims=True)).T
    idx_ref[...] = top_idx.T

    fe = jnp.exp(lt - top_vals[0:1])
    rp = fe / jnp.sum(fe, axis=0, keepdims=True)
    rowsum = jnp.sum(rp, axis=1, keepdims=True)

    @pl.when(i == 0)
    def _init():
        acc_ref[...] = rowsum

    @pl.when(i > 0)
    def _accum():
        acc_ref[...] += rowsum

    @pl.when(i == num_blocks - 1)
    def _finalize():
        ep = acc_ref[...] * (1.0 / total_tokens)
        loss_ref[...] = LB_COEF * jnp.sum(
            ep * jnp.log(ep + 1e-8), keepdims=True)[:, :1]


@functools.partial(jax.jit, static_argnames=())
def kernel(x, W):
    b, s, d = x.shape
    n_tok = b * s
    block_t = 1024
    num_blocks = n_tok // block_t
    x2 = x.reshape(n_tok, d)
    w_bf = W.astype(jnp.bfloat16)

    grid_spec = pl.GridSpec(
        grid=(num_blocks,),
        in_specs=[
            pl.BlockSpec((block_t, d), lambda i: (i, 0)),
            pl.BlockSpec((d, NUM_EXPERTS), lambda i: (0, 0)),
        ],
        out_specs=[
            pl.BlockSpec((block_t, TOP_K), lambda i: (i, 0)),
            pl.BlockSpec((block_t, TOP_K), lambda i: (i, 0)),
            pl.BlockSpec((NUM_EXPERTS, 1), lambda i: (0, 0)),
            pl.BlockSpec((1, 1), lambda i: (0, 0)),
        ],
    )

    probs, idx, _, loss = pl.pallas_call(
        functools.partial(_router_kernel, num_blocks=num_blocks,
                          total_tokens=n_tok),
        grid_spec=grid_spec,
        out_shape=[
            jax.ShapeDtypeStruct((n_tok, TOP_K), jnp.float32),
            jax.ShapeDtypeStruct((n_tok, TOP_K), jnp.int32),
            jax.ShapeDtypeStruct((NUM_EXPERTS, 1), jnp.float32),
            jax.ShapeDtypeStruct((1, 1), jnp.float32),
        ],
    )(x2, w_bf)

    return (probs.reshape(b, s, TOP_K), idx.reshape(b, s, TOP_K),
            loss.reshape(()))
